# baseline (speedup 1.0000x reference)
import functools

import jax
import jax.numpy as jnp
from jax import lax
from jax.experimental import pallas as pl
from jax.experimental.pallas import tpu as pltpu
from jax.experimental.pallas import tpu_sc as plsc

N_RNA = 2000
N_DIS = 1500
N_NODES = N_RNA + N_DIS

_NC = 2
_NS = 16
_NW = _NC * _NS

_CHUNK = 128
_EPT_CHUNKS = 25
_EPT = _CHUNK * _EPT_CHUNKS
_EPAD = _EPT * _NS
_ROWS_PER_SC = N_RNA // _NC
_HALF = _ROWS_PER_SC * N_DIS
_STRIPE = 93752
_LAST_STRIPE = _HALF - (_NS - 1) * _STRIPE
_DUMMY = _HALF

_GB = 3584


def _build_counts(rna_p, dis_p, zeros_stripe, ones_chunk):
    mesh = plsc.VectorSubcoreMesh(core_axis_name="c", subcore_axis_name="s")

    @functools.partial(
        pl.kernel,
        out_type=jax.ShapeDtypeStruct((N_RNA * N_DIS,), jnp.float32),
        mesh=mesh,
        scratch_types=[
            pltpu.VMEM((_EPT,), jnp.int32),
            pltpu.VMEM((_EPT,), jnp.int32),
            pltpu.VMEM((_EPT_CHUNKS, _CHUNK), jnp.int32),
            pltpu.VMEM((_CHUNK,), jnp.float32),
            pltpu.VMEM_SHARED((_HALF + 8,), jnp.float32),
            pltpu.SemaphoreType.DMA,
        ],
        compiler_params=pltpu.CompilerParams(use_tc_tiling_on_sc=False),
    )
    def k(rna_hbm, dis_hbm, z_hbm, ones_hbm, out_hbm,
          rna_v, dis_v, idx_v, val_v, cpart, sem):
        sc = lax.axis_index("c")
        t = lax.axis_index("s")
        off = t * _STRIPE

        @pl.when(t < _NS - 1)
        def _():
            pltpu.async_copy(z_hbm.at[pl.ds(0, _STRIPE)],
                             cpart.at[pl.ds(off, _STRIPE)], sem)

        @pl.when(t == _NS - 1)
        def _():
            pltpu.async_copy(z_hbm.at[pl.ds(0, _LAST_STRIPE + 8)],
                             cpart.at[pl.ds(off, _LAST_STRIPE + 8)], sem)

        base = t * _EPT
        pltpu.sync_copy(rna_hbm.at[pl.ds(base, _EPT)], rna_v)
        pltpu.sync_copy(dis_hbm.at[pl.ds(base, _EPT)], dis_v)
        pltpu.sync_copy(ones_hbm, val_v)

        lo = sc * _ROWS_PER_SC

        def compute_chunk(c, carry):
            for i in range(_CHUNK // 16):
                s = c * _CHUNK + i * 16
                r16 = rna_v[pl.ds(s, 16)]
                d16 = dis_v[pl.ds(s, 16)]
                rr = r16 - lo
                ok = (rr >= 0) & (rr < _ROWS_PER_SC)
                flat = rr * N_DIS + d16
                idx_v[c, pl.ds(i * 16, 16)] = jnp.where(ok, flat, _DUMMY)
            return carry

        lax.fori_loop(0, _EPT_CHUNKS, compute_chunk, 0)

        @pl.when(t < _NS - 1)
        def _():
            pltpu.make_async_copy(z_hbm.at[pl.ds(0, _STRIPE)],
                                  cpart.at[pl.ds(off, _STRIPE)], sem).wait()

        @pl.when(t == _NS - 1)
        def _():
            pltpu.make_async_copy(z_hbm.at[pl.ds(0, _LAST_STRIPE + 8)],
                                  cpart.at[pl.ds(off, _LAST_STRIPE + 8)],
                                  sem).wait()

        plsc.subcore_barrier()

        descs = []
        for c in range(_EPT_CHUNKS):
            descs.append(pltpu.async_copy(
                val_v, cpart.at[idx_v.at[c]], sem, add=True))
        for d in descs:
            d.wait()

        plsc.subcore_barrier()

        obase = sc * _HALF + off

        @pl.when(t < _NS - 1)
        def _():
            pltpu.sync_copy(cpart.at[pl.ds(off, _STRIPE)],
                            out_hbm.at[pl.ds(obase, _STRIPE)])

        @pl.when(t == _NS - 1)
        def _():
            pltpu.sync_copy(cpart.at[pl.ds(off, _LAST_STRIPE)],
                            out_hbm.at[pl.ds(obase, _LAST_STRIPE)])

    return k(rna_p, dis_p, zeros_stripe, ones_chunk)


def _avg_idx_rows(C, S, diagS):
    def body(c_ref, s_ref, dg_ref, o_ref):
        Cm = c_ref[...]
        Y = jnp.dot(Cm, s_ref[...], preferred_element_type=jnp.float32)
        quad = jnp.sum(Y * Cm, axis=1)
        diag_term = jnp.dot(Cm.astype(jnp.bfloat16),
                            dg_ref[...].astype(jnp.bfloat16),
                            preferred_element_type=jnp.float32)
        L = jnp.sum(Cm, axis=1)
        pair_sum = (quad - diag_term) / 2.0
        n_pairs = L * (L - 1.0) / 2.0
        avg = jnp.where(n_pairs > 0, pair_sum / jnp.maximum(n_pairs, 1.0), 0.0)
        o_ref[...] = (avg * 1000.0).astype(jnp.int32)

    return pl.pallas_call(
        body,
        out_shape=jax.ShapeDtypeStruct((C.shape[0],), jnp.int32),
    )(C, S, diagS)


def _avg_idx_cols(C, S, diagS):
    def body(c_ref, s_ref, dg_ref, o_ref):
        Cm = c_ref[...]
        U = jnp.dot(s_ref[...], Cm, preferred_element_type=jnp.float32)
        quad = jnp.sum(Cm * U, axis=0)
        diag_term = lax.dot_general(Cm.astype(jnp.bfloat16),
                                    dg_ref[...].astype(jnp.bfloat16),
                                    (((0,), (0,)), ((), ())),
                                    preferred_element_type=jnp.float32)
        L = jnp.sum(Cm, axis=0)
        pair_sum = (quad - diag_term) / 2.0
        n_pairs = L * (L - 1.0) / 2.0
        avg = jnp.where(n_pairs > 0, pair_sum / jnp.maximum(n_pairs, 1.0), 0.0)
        o_ref[...] = (avg * 1000.0).astype(jnp.int32)

    return pl.pallas_call(
        body,
        out_shape=jax.ShapeDtypeStruct((C.shape[1],), jnp.int32),
    )(C, S, diagS)


def _gather_rows(table, idxp):
    B = idxp.shape[0]
    D = table.shape[1]
    b_per_w = B // _NW
    mesh = plsc.VectorSubcoreMesh(core_axis_name="c", subcore_axis_name="s")

    @functools.partial(
        pl.kernel,
        out_type=jax.ShapeDtypeStruct((B, D), jnp.float32),
        mesh=mesh,
        scratch_types=[
            pltpu.VMEM((b_per_w,), jnp.int32),
            pltpu.VMEM((b_per_w, D), jnp.float32),
            pltpu.SemaphoreType.DMA,
        ],
        compiler_params=pltpu.CompilerParams(use_tc_tiling_on_sc=False),
    )
    def k(table_hbm, idx_hbm, out_hbm, idx_v, rows_v, sem):
        wid = lax.axis_index("s") * _NC + lax.axis_index("c")
        base = wid * b_per_w
        pltpu.sync_copy(idx_hbm.at[pl.ds(base, b_per_w)], idx_v)
        pltpu.async_copy(table_hbm.at[idx_v], rows_v, sem).wait()
        pltpu.sync_copy(rows_v, out_hbm.at[pl.ds(base, b_per_w)])

    return k(table, idxp)


def kernel(associations, ms, ds, emb):
    ne = associations.shape[1]
    rna = associations[0]
    dis = associations[1] - N_RNA
    rna_p = jnp.concatenate(
        [rna, jnp.full((_EPAD - ne,), N_RNA, jnp.int32)])
    dis_p = jnp.concatenate([dis, jnp.zeros((_EPAD - ne,), jnp.int32)])
    zeros_stripe = jnp.zeros((_STRIPE,), jnp.float32)
    ones_chunk = jnp.ones((_CHUNK,), jnp.float32)

    C = _build_counts(rna_p, dis_p, zeros_stripe,
                      ones_chunk).reshape(N_RNA, N_DIS)

    idx_r = _avg_idx_rows(C, ds, jnp.diagonal(ds))
    idx_d = _avg_idx_cols(C, ms, jnp.diagonal(ms))
    idx = jnp.concatenate([idx_r, idx_d])
    idx_p = jnp.concatenate([idx, jnp.zeros((_GB - N_NODES,), jnp.int32)])

    out = _gather_rows(emb, idx_p)
    return out[:N_NODES]


# speedup vs baseline: 1.0083x; 1.0083x over previous
import functools

import jax
import jax.numpy as jnp
from jax import lax
from jax.experimental import pallas as pl
from jax.experimental.pallas import tpu as pltpu
from jax.experimental.pallas import tpu_sc as plsc

N_RNA = 2000
N_DIS = 1500
N_NODES = N_RNA + N_DIS

_NC = 2
_NS = 16
_NW = _NC * _NS

_CHUNK = 128
_EPT_CHUNKS = 25
_EPT = _CHUNK * _EPT_CHUNKS
_EPAD = _EPT * _NS
_ROWS_PER_SC = N_RNA // _NC
_HALF = _ROWS_PER_SC * N_DIS
_STRIPE = 93752
_LAST_STRIPE = _HALF - (_NS - 1) * _STRIPE
_DUMMY = _HALF

_GB = 3584


def _build_counts(rna_p, dis_p, zeros_stripe, ones_chunk):
    mesh = plsc.VectorSubcoreMesh(core_axis_name="c", subcore_axis_name="s")

    @functools.partial(
        pl.kernel,
        out_type=jax.ShapeDtypeStruct((N_RNA * N_DIS,), jnp.float32),
        mesh=mesh,
        scratch_types=[
            pltpu.VMEM((_EPT,), jnp.int32),
            pltpu.VMEM((_EPT,), jnp.int32),
            pltpu.VMEM((_EPT_CHUNKS, _CHUNK), jnp.int32),
            pltpu.VMEM((_CHUNK,), jnp.float32),
            pltpu.VMEM_SHARED((_HALF + 2048,), jnp.float32),
            pltpu.SemaphoreType.DMA,
        ],
        compiler_params=pltpu.CompilerParams(use_tc_tiling_on_sc=False),
    )
    def k(rna_hbm, dis_hbm, z_hbm, ones_hbm, out_hbm,
          rna_v, dis_v, idx_v, val_v, cpart, sem):
        sc = lax.axis_index("c")
        t = lax.axis_index("s")
        off = t * _STRIPE

        @pl.when(t < _NS - 1)
        def _():
            pltpu.async_copy(z_hbm.at[pl.ds(0, _STRIPE)],
                             cpart.at[pl.ds(off, _STRIPE)], sem)

        @pl.when(t == _NS - 1)
        def _():
            pltpu.async_copy(z_hbm.at[pl.ds(0, _LAST_STRIPE + 2048)],
                             cpart.at[pl.ds(off, _LAST_STRIPE + 2048)], sem)

        base = t * _EPT
        pltpu.sync_copy(rna_hbm.at[pl.ds(base, _EPT)], rna_v)
        pltpu.sync_copy(dis_hbm.at[pl.ds(base, _EPT)], dis_v)
        pltpu.sync_copy(ones_hbm, val_v)

        lo = sc * _ROWS_PER_SC

        def compute_chunk(c, carry):
            for i in range(_CHUNK // 16):
                s = c * _CHUNK + i * 16
                r16 = rna_v[pl.ds(s, 16)]
                d16 = dis_v[pl.ds(s, 16)]
                rr = r16 - lo
                ok = (rr >= 0) & (rr < _ROWS_PER_SC)
                flat = rr * N_DIS + d16
                idx_v[c, pl.ds(i * 16, 16)] = jnp.where(ok, flat, _DUMMY + d16)
            return carry

        lax.fori_loop(0, _EPT_CHUNKS, compute_chunk, 0)

        @pl.when(t < _NS - 1)
        def _():
            pltpu.make_async_copy(z_hbm.at[pl.ds(0, _STRIPE)],
                                  cpart.at[pl.ds(off, _STRIPE)], sem).wait()

        @pl.when(t == _NS - 1)
        def _():
            pltpu.make_async_copy(z_hbm.at[pl.ds(0, _LAST_STRIPE + 2048)],
                                  cpart.at[pl.ds(off, _LAST_STRIPE + 2048)],
                                  sem).wait()

        plsc.subcore_barrier()

        descs = []
        for c in range(_EPT_CHUNKS):
            descs.append(pltpu.async_copy(
                val_v, cpart.at[idx_v.at[c]], sem, add=True))
        for d in descs:
            d.wait()

        plsc.subcore_barrier()

        obase = sc * _HALF + off

        @pl.when(t < _NS - 1)
        def _():
            pltpu.sync_copy(cpart.at[pl.ds(off, _STRIPE)],
                            out_hbm.at[pl.ds(obase, _STRIPE)])

        @pl.when(t == _NS - 1)
        def _():
            pltpu.sync_copy(cpart.at[pl.ds(off, _LAST_STRIPE)],
                            out_hbm.at[pl.ds(obase, _LAST_STRIPE)])

    return k(rna_p, dis_p, zeros_stripe, ones_chunk)


def _avg_idx_rows(C, S, diagS):
    def body(c_ref, s_ref, dg_ref, o_ref):
        Cm = c_ref[...]
        Y = jnp.dot(Cm, s_ref[...], preferred_element_type=jnp.float32)
        quad = jnp.sum(Y * Cm, axis=1)
        diag_term = jnp.dot(Cm.astype(jnp.bfloat16),
                            dg_ref[...].astype(jnp.bfloat16),
                            preferred_element_type=jnp.float32)
        L = jnp.sum(Cm, axis=1)
        pair_sum = (quad - diag_term) / 2.0
        n_pairs = L * (L - 1.0) / 2.0
        avg = jnp.where(n_pairs > 0, pair_sum / jnp.maximum(n_pairs, 1.0), 0.0)
        o_ref[...] = (avg * 1000.0).astype(jnp.int32)

    return pl.pallas_call(
        body,
        out_shape=jax.ShapeDtypeStruct((C.shape[0],), jnp.int32),
    )(C, S, diagS)


def _avg_idx_cols(C, S, diagS):
    def body(c_ref, s_ref, dg_ref, o_ref):
        Cm = c_ref[...]
        U = jnp.dot(s_ref[...], Cm, preferred_element_type=jnp.float32)
        quad = jnp.sum(Cm * U, axis=0)
        diag_term = lax.dot_general(Cm.astype(jnp.bfloat16),
                                    dg_ref[...].astype(jnp.bfloat16),
                                    (((0,), (0,)), ((), ())),
                                    preferred_element_type=jnp.float32)
        L = jnp.sum(Cm, axis=0)
        pair_sum = (quad - diag_term) / 2.0
        n_pairs = L * (L - 1.0) / 2.0
        avg = jnp.where(n_pairs > 0, pair_sum / jnp.maximum(n_pairs, 1.0), 0.0)
        o_ref[...] = (avg * 1000.0).astype(jnp.int32)

    return pl.pallas_call(
        body,
        out_shape=jax.ShapeDtypeStruct((C.shape[1],), jnp.int32),
    )(C, S, diagS)


def _gather_rows(table, idxp):
    B = idxp.shape[0]
    D = table.shape[1]
    b_per_w = B // _NW
    mesh = plsc.VectorSubcoreMesh(core_axis_name="c", subcore_axis_name="s")

    @functools.partial(
        pl.kernel,
        out_type=jax.ShapeDtypeStruct((B, D), jnp.float32),
        mesh=mesh,
        scratch_types=[
            pltpu.VMEM((b_per_w,), jnp.int32),
            pltpu.VMEM((b_per_w, D), jnp.float32),
            pltpu.SemaphoreType.DMA,
        ],
        compiler_params=pltpu.CompilerParams(use_tc_tiling_on_sc=False),
    )
    def k(table_hbm, idx_hbm, out_hbm, idx_v, rows_v, sem):
        wid = lax.axis_index("s") * _NC + lax.axis_index("c")
        base = wid * b_per_w
        pltpu.sync_copy(idx_hbm.at[pl.ds(base, b_per_w)], idx_v)
        pltpu.async_copy(table_hbm.at[idx_v], rows_v, sem).wait()
        pltpu.sync_copy(rows_v, out_hbm.at[pl.ds(base, b_per_w)])

    return k(table, idxp)


def kernel(associations, ms, ds, emb):
    ne = associations.shape[1]
    rna = associations[0]
    dis = associations[1] - N_RNA
    rna_p = jnp.concatenate(
        [rna, jnp.full((_EPAD - ne,), N_RNA, jnp.int32)])
    dis_p = jnp.concatenate([dis, jnp.zeros((_EPAD - ne,), jnp.int32)])
    zeros_stripe = jnp.zeros((_LAST_STRIPE + 2048,), jnp.float32)
    ones_chunk = jnp.ones((_CHUNK,), jnp.float32)

    C = _build_counts(rna_p, dis_p, zeros_stripe,
                      ones_chunk).reshape(N_RNA, N_DIS)

    idx_r = _avg_idx_rows(C, ds, jnp.diagonal(ds))
    idx_d = _avg_idx_cols(C, ms, jnp.diagonal(ms))
    idx = jnp.concatenate([idx_r, idx_d])
    idx_p = jnp.concatenate([idx, jnp.zeros((_GB - N_NODES,), jnp.int32)])

    out = _gather_rows(emb, idx_p)
    return out[:N_NODES]


# speedup vs baseline: 1.0946x; 1.0855x over previous
import functools

import jax
import jax.numpy as jnp
from jax import lax
from jax.experimental import pallas as pl
from jax.experimental.pallas import tpu as pltpu
from jax.experimental.pallas import tpu_sc as plsc

N_RNA = 2000
N_DIS = 1500
N_NODES = N_RNA + N_DIS

_NC = 2
_NS = 16
_NW = _NC * _NS

_CHUNK = 128
_EPT_CHUNKS = 25
_EPT = _CHUNK * _EPT_CHUNKS
_EPAD = _EPT * _NS
_ROWS_PER_SC = N_RNA // _NC
_HALF = _ROWS_PER_SC * N_DIS
_STRIPE = 93752
_LAST_STRIPE = _HALF - (_NS - 1) * _STRIPE
_DUMMY = _HALF

_GB = 3584


def _build_counts(rna_p, dis_p, zeros_stripe, ones_chunk):
    mesh = plsc.VectorSubcoreMesh(core_axis_name="c", subcore_axis_name="s")

    @functools.partial(
        pl.kernel,
        out_type=jax.ShapeDtypeStruct((N_RNA * N_DIS,), jnp.float32),
        mesh=mesh,
        scratch_types=[
            pltpu.VMEM((_EPT,), jnp.int32),
            pltpu.VMEM((_EPT,), jnp.int32),
            pltpu.VMEM((_EPT_CHUNKS, _CHUNK), jnp.int32),
            pltpu.VMEM((_CHUNK,), jnp.float32),
            pltpu.VMEM_SHARED((_HALF + 2048,), jnp.float32),
            pltpu.SemaphoreType.DMA,
        ],
        compiler_params=pltpu.CompilerParams(use_tc_tiling_on_sc=False),
    )
    def k(rna_hbm, dis_hbm, z_hbm, ones_hbm, out_hbm,
          rna_v, dis_v, idx_v, val_v, cpart, sem):
        sc = lax.axis_index("c")
        t = lax.axis_index("s")
        off = t * _STRIPE

        @pl.when(t < _NS - 1)
        def _():
            pltpu.async_copy(z_hbm.at[pl.ds(0, _STRIPE)],
                             cpart.at[pl.ds(off, _STRIPE)], sem)

        @pl.when(t == _NS - 1)
        def _():
            pltpu.async_copy(z_hbm.at[pl.ds(0, _LAST_STRIPE + 2048)],
                             cpart.at[pl.ds(off, _LAST_STRIPE + 2048)], sem)

        base = t * _EPT
        pltpu.sync_copy(rna_hbm.at[pl.ds(base, _EPT)], rna_v)
        pltpu.sync_copy(dis_hbm.at[pl.ds(base, _EPT)], dis_v)
        pltpu.sync_copy(ones_hbm, val_v)

        lo = sc * _ROWS_PER_SC

        def compute_chunk(c, carry):
            for i in range(_CHUNK // 16):
                s = c * _CHUNK + i * 16
                r16 = rna_v[pl.ds(s, 16)]
                d16 = dis_v[pl.ds(s, 16)]
                rr = r16 - lo
                ok = (rr >= 0) & (rr < _ROWS_PER_SC)
                flat = rr * N_DIS + d16
                idx_v[c, pl.ds(i * 16, 16)] = jnp.where(ok, flat, _DUMMY + d16)
            return carry

        lax.fori_loop(0, _EPT_CHUNKS, compute_chunk, 0)

        @pl.when(t < _NS - 1)
        def _():
            pltpu.make_async_copy(z_hbm.at[pl.ds(0, _STRIPE)],
                                  cpart.at[pl.ds(off, _STRIPE)], sem).wait()

        @pl.when(t == _NS - 1)
        def _():
            pltpu.make_async_copy(z_hbm.at[pl.ds(0, _LAST_STRIPE + 2048)],
                                  cpart.at[pl.ds(off, _LAST_STRIPE + 2048)],
                                  sem).wait()

        plsc.subcore_barrier()

        descs = []
        for c in range(_EPT_CHUNKS):
            descs.append(pltpu.async_copy(
                val_v, cpart.at[idx_v.at[c]], sem, add=True))
        for d in descs:
            d.wait()

        plsc.subcore_barrier()

        obase = sc * _HALF + off

        @pl.when(t < _NS - 1)
        def _():
            pltpu.sync_copy(cpart.at[pl.ds(off, _STRIPE)],
                            out_hbm.at[pl.ds(obase, _STRIPE)])

        @pl.when(t == _NS - 1)
        def _():
            pltpu.sync_copy(cpart.at[pl.ds(off, _LAST_STRIPE)],
                            out_hbm.at[pl.ds(obase, _LAST_STRIPE)])

    return k(rna_p, dis_p, zeros_stripe, ones_chunk)


_EMB_K = 1024


def _avg_and_lookup(C, ms, ds, dgds, dgms, et):
    def body(c_ref, ms_ref, ds_ref, dgds_ref, dgms_ref, et_ref, o_ref):
        Cm = c_ref[...]
        Cb = Cm.astype(jnp.bfloat16)

        Y = jnp.dot(Cm, ds_ref[...], preferred_element_type=jnp.float32)
        quad_r = jnp.sum(Y * Cm, axis=1)
        dterm_r = jnp.dot(Cb, dgds_ref[...].astype(jnp.bfloat16),
                          preferred_element_type=jnp.float32)
        L_r = jnp.sum(Cm, axis=1)
        pair_r = (quad_r - dterm_r) / 2.0
        np_r = L_r * (L_r - 1.0) / 2.0
        avg_r = jnp.where(np_r > 0, pair_r / jnp.maximum(np_r, 1.0), 0.0)

        U = jnp.dot(ms_ref[...], Cm, preferred_element_type=jnp.float32)
        quad_d = jnp.sum(Cm * U, axis=0)
        dterm_d = lax.dot_general(Cb, dgms_ref[...].astype(jnp.bfloat16),
                                  (((0,), (0,)), ((), ())),
                                  preferred_element_type=jnp.float32)
        L_d = jnp.sum(Cm, axis=0)
        pair_d = (quad_d - dterm_d) / 2.0
        np_d = L_d * (L_d - 1.0) / 2.0
        avg_d = jnp.where(np_d > 0, pair_d / jnp.maximum(np_d, 1.0), 0.0)

        idx = (jnp.concatenate([avg_r, avg_d]) * 1000.0).astype(jnp.int32)

        et = et_ref[...]
        ehi = et.astype(jnp.bfloat16)
        r1 = et - ehi.astype(jnp.float32)
        emid = r1.astype(jnp.bfloat16)
        elo = (r1 - emid.astype(jnp.float32)).astype(jnp.bfloat16)
        cols = lax.broadcasted_iota(jnp.int32, (N_NODES, _EMB_K), 1)
        onehot = (cols == idx[:, None]).astype(jnp.bfloat16)
        g_hi = jnp.dot(onehot, ehi, preferred_element_type=jnp.float32)
        g_mid = jnp.dot(onehot, emid, preferred_element_type=jnp.float32)
        g_lo = jnp.dot(onehot, elo, preferred_element_type=jnp.float32)
        o_ref[...] = g_hi + (g_mid + g_lo)

    return pl.pallas_call(
        body,
        out_shape=jax.ShapeDtypeStruct((N_NODES, et.shape[1]), jnp.float32),
    )(C, ms, ds, dgds, dgms, et)


def kernel(associations, ms, ds, emb):
    ne = associations.shape[1]
    rna = associations[0]
    dis = associations[1] - N_RNA
    rna_p = jnp.concatenate(
        [rna, jnp.full((_EPAD - ne,), N_RNA, jnp.int32)])
    dis_p = jnp.concatenate([dis, jnp.zeros((_EPAD - ne,), jnp.int32)])
    zeros_stripe = jnp.zeros((_LAST_STRIPE + 2048,), jnp.float32)
    ones_chunk = jnp.ones((_CHUNK,), jnp.float32)

    C = _build_counts(rna_p, dis_p, zeros_stripe,
                      ones_chunk).reshape(N_RNA, N_DIS)

    return _avg_and_lookup(C, ms, ds, jnp.diagonal(ds), jnp.diagonal(ms),
                           emb[:_EMB_K])
